# baseline (speedup 1.0000x reference)
import functools
import math

import jax
import jax.numpy as jnp
from jax import lax
from jax.experimental import pallas as pl
from jax.experimental.pallas import tpu as pltpu
from jax.experimental.pallas import tpu_sc as plsc

MARGIN1 = 0.3
MARGIN2 = 0.3

NC = 2
NS = 16
NW = NC * NS
L = 16
TILE = 128


def _sc_triplet_partials(emb, ia, ip, ir, inn, *, t_valid, steps):
    d = emb.shape[1]
    pw = steps * TILE
    mesh = plsc.VectorSubcoreMesh(
        core_axis_name="c", subcore_axis_name="s",
        num_cores=NC, num_subcores=NS)

    @functools.partial(
        pl.kernel,
        out_type=jax.ShapeDtypeStruct((NW, L), jnp.float32),
        mesh=mesh,
        scratch_types=[
            pltpu.VMEM((TILE,), jnp.int32),
            pltpu.VMEM((TILE,), jnp.int32),
            pltpu.VMEM((TILE,), jnp.int32),
            pltpu.VMEM((TILE,), jnp.int32),
            pltpu.VMEM((TILE, d), jnp.float32),
            pltpu.VMEM((TILE, d), jnp.float32),
            pltpu.VMEM((TILE, d), jnp.float32),
            pltpu.VMEM((TILE, d), jnp.float32),
            pltpu.VMEM((L,), jnp.float32),
            pltpu.SemaphoreType.DMA,
        ],
        compiler_params=pltpu.CompilerParams(needs_layout_passes=False),
    )
    def body(emb_hbm, ia_hbm, ip_hbm, ir_hbm, in_hbm, out_hbm,
             ia_v, ip_v, ir_v, in_v, ra_v, rp_v, rr_v, rn_v, acc_v, sem):
        wid = lax.axis_index("s") * NC + lax.axis_index("c")
        base_w = wid * pw
        lanes = lax.iota(jnp.int32, L)
        zero = jnp.zeros((L,), jnp.float32)

        def step(s, acc):
            base = base_w + s * TILE
            pltpu.sync_copy(ia_hbm.at[pl.ds(base, TILE)], ia_v)
            pltpu.sync_copy(ip_hbm.at[pl.ds(base, TILE)], ip_v)
            pltpu.sync_copy(ir_hbm.at[pl.ds(base, TILE)], ir_v)
            pltpu.sync_copy(in_hbm.at[pl.ds(base, TILE)], in_v)
            cps = [
                pltpu.async_copy(emb_hbm.at[ia_v], ra_v, sem),
                pltpu.async_copy(emb_hbm.at[ip_v], rp_v, sem),
                pltpu.async_copy(emb_hbm.at[ir_v], rr_v, sem),
                pltpu.async_copy(emb_hbm.at[in_v], rn_v, sem),
            ]
            for cp in cps:
                cp.wait()

            def group(g, acc):
                rows = g * L + lanes

                def dstep(dd, accs):
                    aap, aar, aan = accs
                    cols = jnp.full((L,), dd, jnp.int32)
                    av = plsc.load_gather(ra_v, [rows, cols])
                    pv = plsc.load_gather(rp_v, [rows, cols])
                    rv = plsc.load_gather(rr_v, [rows, cols])
                    nv = plsc.load_gather(rn_v, [rows, cols])
                    dap = av - pv
                    dar = av - rv
                    dan = av - nv
                    return (aap + dap * dap, aar + dar * dar, aan + dan * dan)

                aap, aar, aan = lax.fori_loop(
                    0, d, dstep, (zero, zero, zero), unroll=8)
                losses = (jnp.maximum(aap - aar + MARGIN1, 0.0)
                          + jnp.maximum(aar - aan + MARGIN2, 0.0))
                gt = base + g * L + lanes
                return acc + jnp.where(gt < t_valid, losses, 0.0)

            return lax.fori_loop(0, TILE // L, group, acc)

        acc = lax.fori_loop(0, steps, step, zero)
        acc_v[...] = acc
        pltpu.sync_copy(acc_v, out_hbm.at[wid])

    return body(emb, ia, ip, ir, inn)


def kernel(embeddings, target, triplets):
    del target
    t = triplets.shape[0]
    steps = math.ceil(t / (NW * TILE))
    tpad = NW * TILE * steps
    tri = triplets.astype(jnp.int32)
    tri = jnp.pad(tri, ((0, tpad - t), (0, 0)))
    parts = _sc_triplet_partials(
        embeddings, tri[:, 0], tri[:, 1], tri[:, 2], tri[:, 3],
        t_valid=t, steps=steps)
    loss = jnp.sum(parts) / jnp.float32(t)
    return (loss, jnp.asarray(t, dtype=jnp.int32))


# speedup vs baseline: 1.8951x; 1.8951x over previous
import functools
import math

import jax
import jax.numpy as jnp
from jax import lax
from jax.experimental import pallas as pl
from jax.experimental.pallas import tpu as pltpu
from jax.experimental.pallas import tpu_sc as plsc

MARGIN1 = 0.3
MARGIN2 = 0.3

NC = 2
NS = 16
NW = NC * NS
L = 16
TILE = 112


def _sc_triplet_partials(emb, idx_slab, *, t_valid, steps):
    d = emb.shape[1]
    pw = steps * TILE
    r4 = 4 * TILE
    mesh = plsc.VectorSubcoreMesh(
        core_axis_name="c", subcore_axis_name="s",
        num_cores=NC, num_subcores=NS)

    @functools.partial(
        pl.kernel,
        out_type=jax.ShapeDtypeStruct((NW, L), jnp.float32),
        mesh=mesh,
        scratch_types=[
            pltpu.VMEM((steps, 4, TILE), jnp.int32),
            pltpu.VMEM((r4, d), jnp.float32),
            pltpu.VMEM((r4, d), jnp.float32),
            pltpu.VMEM((L,), jnp.float32),
            pltpu.SemaphoreType.DMA,
            pltpu.SemaphoreType.DMA,
        ],
        compiler_params=pltpu.CompilerParams(needs_layout_passes=False),
    )
    def body(emb_hbm, idx_hbm, out_hbm, idx_v, rows0_v, rows1_v, acc_v,
             sem0, sem1):
        wid = lax.axis_index("s") * NC + lax.axis_index("c")
        base_w = wid * pw
        lanes = lax.iota(jnp.int32, L)
        zero = jnp.zeros((L,), jnp.float32)

        pltpu.sync_copy(idx_hbm.at[wid], idx_v)

        def issue(s, rows_v, sem):
            for j in range(4):
                pltpu.async_copy(emb_hbm.at[idx_v.at[s, j]],
                                 rows_v.at[pl.ds(j * TILE, TILE)], sem)

        def drain(rows_v, sem):
            for j in range(4):
                pltpu.make_async_copy(emb_hbm.at[idx_v.at[0, 0]],
                                      rows_v.at[pl.ds(j * TILE, TILE)],
                                      sem).wait()

        issue(0, rows0_v, sem0)

        def compute(rows_v, s, acc):
            base = base_w + s * TILE

            def group(g, acc):
                rows = g * L + lanes

                def dstep(dd, accs):
                    aap, aar, aan = accs
                    cols = jnp.full((L,), dd, jnp.int32)
                    av = plsc.load_gather(rows_v, [rows, cols])
                    pv = plsc.load_gather(rows_v, [rows + TILE, cols])
                    rv = plsc.load_gather(rows_v, [rows + 2 * TILE, cols])
                    nv = plsc.load_gather(rows_v, [rows + 3 * TILE, cols])
                    dap = av - pv
                    dar = av - rv
                    dan = av - nv
                    return (aap + dap * dap, aar + dar * dar, aan + dan * dan)

                aap, aar, aan = lax.fori_loop(
                    0, d, dstep, (zero, zero, zero), unroll=8)
                losses = (jnp.maximum(aap - aar + MARGIN1, 0.0)
                          + jnp.maximum(aar - aan + MARGIN2, 0.0))
                gt = base + g * L + lanes
                return acc + jnp.where(gt < t_valid, losses, 0.0)

            return lax.fori_loop(0, TILE // L, group, acc)

        def double_step(i, acc):
            s0 = 2 * i
            s1 = s0 + 1
            drain(rows0_v, sem0)
            issue(s1, rows1_v, sem1)
            acc = compute(rows0_v, s0, acc)
            drain(rows1_v, sem1)

            @pl.when(s1 + 1 < steps)
            def _():
                issue(s1 + 1, rows0_v, sem0)

            return compute(rows1_v, s1, acc)

        acc = lax.fori_loop(0, steps // 2, double_step, zero)
        acc_v[...] = acc
        pltpu.sync_copy(acc_v, out_hbm.at[wid])

    return body(emb, idx_slab)


def kernel(embeddings, target, triplets):
    del target
    t = triplets.shape[0]
    steps = math.ceil(t / (NW * TILE))
    steps += steps % 2
    tpad = NW * TILE * steps
    tri = triplets.astype(jnp.int32)
    tri = jnp.pad(tri, ((0, tpad - t), (0, 0)))
    idx_slab = tri.reshape(NW, steps, TILE, 4).transpose(0, 1, 3, 2)
    parts = _sc_triplet_partials(
        embeddings, idx_slab, t_valid=t, steps=steps)
    loss = jnp.sum(parts) / jnp.float32(t)
    return (loss, jnp.asarray(t, dtype=jnp.int32))


# speedup vs baseline: 7.3724x; 3.8902x over previous
import functools
import math

import jax
import jax.numpy as jnp
from jax import lax
from jax.experimental import pallas as pl
from jax.experimental.pallas import tpu as pltpu
from jax.experimental.pallas import tpu_sc as plsc

MARGIN1 = 0.3
MARGIN2 = 0.3

NC = 2
NS = 16
NW = NC * NS
L = 16
TILE = 112


def _sc_triplet_partials(emb, idx_slab, *, t_valid, steps):
    d = emb.shape[1]
    pw = steps * TILE
    r4 = 4 * TILE
    mesh = plsc.VectorSubcoreMesh(
        core_axis_name="c", subcore_axis_name="s",
        num_cores=NC, num_subcores=NS)

    @functools.partial(
        pl.kernel,
        out_type=jax.ShapeDtypeStruct((NW, L), jnp.float32),
        mesh=mesh,
        scratch_types=[
            pltpu.VMEM((steps, 4, TILE), jnp.int32),
            pltpu.VMEM((r4, d), jnp.float32),
            pltpu.VMEM((r4, d), jnp.float32),
            pltpu.VMEM((L,), jnp.float32),
            pltpu.SemaphoreType.DMA,
            pltpu.SemaphoreType.DMA,
        ],
        compiler_params=pltpu.CompilerParams(needs_layout_passes=False),
    )
    def body(emb_hbm, idx_hbm, out_hbm, idx_v, rows0_v, rows1_v, acc_v,
             sem0, sem1):
        wid = lax.axis_index("s") * NC + lax.axis_index("c")
        base_w = wid * pw
        lanes = lax.iota(jnp.int32, L)
        zero = jnp.zeros((L,), jnp.float32)

        pltpu.sync_copy(idx_hbm.at[wid], idx_v)

        def issue(s, rows_v, sem):
            for j in range(4):
                pltpu.async_copy(emb_hbm.at[idx_v.at[s, j]],
                                 rows_v.at[pl.ds(j * TILE, TILE)], sem)

        def drain(rows_v, sem):
            for j in range(4):
                pltpu.make_async_copy(emb_hbm.at[idx_v.at[0, 0]],
                                      rows_v.at[pl.ds(j * TILE, TILE)],
                                      sem).wait()

        issue(0, rows0_v, sem0)

        def compute(rows_v, s, acc):
            base = base_w + s * TILE

            def group(g, acc):
                rows = g * L + lanes

                def dstep(dd, accs):
                    aap, aar, aan = accs
                    cols = (lanes + dd) & (d - 1)
                    av = plsc.load_gather(rows_v, [rows, cols])
                    pv = plsc.load_gather(rows_v, [rows + TILE, cols])
                    rv = plsc.load_gather(rows_v, [rows + 2 * TILE, cols])
                    nv = plsc.load_gather(rows_v, [rows + 3 * TILE, cols])
                    dap = av - pv
                    dar = av - rv
                    dan = av - nv
                    return (aap + dap * dap, aar + dar * dar, aan + dan * dan)

                aap, aar, aan = lax.fori_loop(
                    0, d, dstep, (zero, zero, zero), unroll=8)
                losses = (jnp.maximum(aap - aar + MARGIN1, 0.0)
                          + jnp.maximum(aar - aan + MARGIN2, 0.0))
                gt = base + g * L + lanes
                return acc + jnp.where(gt < t_valid, losses, 0.0)

            return lax.fori_loop(0, TILE // L, group, acc)

        def double_step(i, acc):
            s0 = 2 * i
            s1 = s0 + 1
            drain(rows0_v, sem0)
            issue(s1, rows1_v, sem1)
            acc = compute(rows0_v, s0, acc)
            drain(rows1_v, sem1)

            @pl.when(s1 + 1 < steps)
            def _():
                issue(s1 + 1, rows0_v, sem0)

            return compute(rows1_v, s1, acc)

        acc = lax.fori_loop(0, steps // 2, double_step, zero)
        acc_v[...] = acc
        pltpu.sync_copy(acc_v, out_hbm.at[wid])

    return body(emb, idx_slab)


def kernel(embeddings, target, triplets):
    del target
    t = triplets.shape[0]
    steps = math.ceil(t / (NW * TILE))
    steps += steps % 2
    tpad = NW * TILE * steps
    tri = triplets.astype(jnp.int32)
    tri = jnp.pad(tri, ((0, tpad - t), (0, 0)))
    idx_slab = tri.reshape(NW, steps, TILE, 4).transpose(0, 1, 3, 2)
    parts = _sc_triplet_partials(
        embeddings, idx_slab, t_valid=t, steps=steps)
    loss = jnp.sum(parts) / jnp.float32(t)
    return (loss, jnp.asarray(t, dtype=jnp.int32))


# speedup vs baseline: 9.1737x; 1.2443x over previous
import functools
import math

import jax
import jax.numpy as jnp
from jax import lax
from jax.experimental import pallas as pl
from jax.experimental.pallas import tpu as pltpu
from jax.experimental.pallas import tpu_sc as plsc

MARGIN1 = 0.3
MARGIN2 = 0.3

NC = 2
NS = 16
NW = NC * NS
L = 16
TILE = 72


def _sc_triplet_partials(emb, idx_slab, *, t_valid, steps):
    d = emb.shape[1]
    pw = steps * TILE
    r4 = 4 * TILE
    mesh = plsc.VectorSubcoreMesh(
        core_axis_name="c", subcore_axis_name="s",
        num_cores=NC, num_subcores=NS)

    @functools.partial(
        pl.kernel,
        out_type=jax.ShapeDtypeStruct((NW, L), jnp.float32),
        mesh=mesh,
        scratch_types=[
            pltpu.VMEM((steps, 4, TILE), jnp.int32),
            pltpu.VMEM((r4, d), jnp.float32),
            pltpu.VMEM((r4, d), jnp.float32),
            pltpu.VMEM((L,), jnp.float32),
            pltpu.VMEM_SHARED((4096, 128), jnp.float32),
            pltpu.SemaphoreType.DMA,
            pltpu.SemaphoreType.DMA,
        ],
        compiler_params=pltpu.CompilerParams(needs_layout_passes=False),
    )
    def body(emb_hbm, idx_hbm, out_hbm, idx_v, rows0_v, rows1_v, acc_v,
             emb_sh, sem0, sem1):
        sid = lax.axis_index("s")
        wid = sid * NC + lax.axis_index("c")
        base_w = wid * pw
        lanes = lax.iota(jnp.int32, L)
        zero = jnp.zeros((L,), jnp.float32)

        rows_per_tile = emb_hbm.shape[0] // NS
        pltpu.sync_copy(emb_hbm.at[pl.ds(sid * rows_per_tile, rows_per_tile)],
                        emb_sh.at[pl.ds(sid * rows_per_tile, rows_per_tile)])
        pltpu.sync_copy(idx_hbm.at[wid], idx_v)
        plsc.subcore_barrier()

        def issue(s, rows_v, sem):
            for j in range(4):
                pltpu.async_copy(emb_sh.at[idx_v.at[s, j]],
                                 rows_v.at[pl.ds(j * TILE, TILE)], sem)

        def drain(rows_v, sem):
            for j in range(4):
                pltpu.make_async_copy(emb_sh.at[idx_v.at[0, 0]],
                                      rows_v.at[pl.ds(j * TILE, TILE)],
                                      sem).wait()

        issue(0, rows0_v, sem0)

        def compute(rows_v, s, acc):
            base = base_w + s * TILE

            def group(g, acc):
                rows = g * L + lanes

                def dstep(dd, accs):
                    aap, aar, aan = accs
                    cols = (lanes + dd) & (d - 1)
                    av = plsc.load_gather(rows_v, [rows, cols])
                    pv = plsc.load_gather(rows_v, [rows + TILE, cols])
                    rv = plsc.load_gather(rows_v, [rows + 2 * TILE, cols])
                    nv = plsc.load_gather(rows_v, [rows + 3 * TILE, cols])
                    dap = av - pv
                    dar = av - rv
                    dan = av - nv
                    return (aap + dap * dap, aar + dar * dar, aan + dan * dan)

                aap, aar, aan = lax.fori_loop(
                    0, d, dstep, (zero, zero, zero), unroll=8)
                losses = (jnp.maximum(aap - aar + MARGIN1, 0.0)
                          + jnp.maximum(aar - aan + MARGIN2, 0.0))
                gt = base + g * L + lanes
                return acc + jnp.where(gt < t_valid, losses, 0.0)

            return lax.fori_loop(0, TILE // L, group, acc)

        def double_step(i, acc):
            s0 = 2 * i
            s1 = s0 + 1
            drain(rows0_v, sem0)
            issue(s1, rows1_v, sem1)
            acc = compute(rows0_v, s0, acc)
            drain(rows1_v, sem1)

            @pl.when(s1 + 1 < steps)
            def _():
                issue(s1 + 1, rows0_v, sem0)

            return compute(rows1_v, s1, acc)

        acc = lax.fori_loop(0, steps // 2, double_step, zero)
        acc_v[...] = acc
        pltpu.sync_copy(acc_v, out_hbm.at[wid])

    return body(emb, idx_slab)


def kernel(embeddings, target, triplets):
    del target
    t = triplets.shape[0]
    steps = math.ceil(t / (NW * TILE))
    steps += steps % 2
    tpad = NW * TILE * steps
    tri = triplets.astype(jnp.int32)
    tri = jnp.pad(tri, ((0, tpad - t), (0, 0)))
    idx_slab = tri.reshape(NW, steps, TILE, 4).transpose(0, 1, 3, 2)
    parts = _sc_triplet_partials(
        embeddings, idx_slab, t_valid=t, steps=steps)
    loss = jnp.sum(parts) / jnp.float32(t)
    return (loss, jnp.asarray(t, dtype=jnp.int32))
